# baseline (speedup 1.0000x reference)
import functools

import jax
import jax.numpy as jnp
from jax import lax
from jax.experimental import pallas as pl
from jax.experimental.pallas import tpu as pltpu
from jax.experimental.pallas import tpu_sc as plsc

D = 128
NSLOT = 11
NC, NS = 2, 16
NW = NC * NS
R = 56
BLK = 512


def _sc_gather_sum(features, idxT, b_pad):
    bpw = b_pad // NW
    nchunks = bpw // R
    mesh = plsc.VectorSubcoreMesh(core_axis_name="c", subcore_axis_name="s")

    @functools.partial(
        pl.kernel,
        out_type=(jax.ShapeDtypeStruct((b_pad, D), jnp.float32),
                  jax.ShapeDtypeStruct((b_pad, D), jnp.float32)),
        mesh=mesh,
        scratch_types=[
            pltpu.VMEM((NSLOT, bpw), jnp.int32),
            pltpu.VMEM((R, D), jnp.float32),
            pltpu.VMEM((R, D), jnp.float32),
            pltpu.SemaphoreType.DMA,
            pltpu.SemaphoreType.DMA,
        ],
        compiler_params=pltpu.CompilerParams(use_tc_tiling_on_sc=False),
    )
    def k(feat_hbm, idxT_hbm, self_hbm, nsum_hbm, idx_v, sbuf, nbuf, gsem,
          nsem):
        wid = lax.axis_index("s") * NC + lax.axis_index("c")
        base = wid * bpw
        pltpu.sync_copy(idxT_hbm.at[wid], idx_v)

        def chunk(ci, carry):
            off = ci * R
            cp_self = pltpu.async_copy(
                feat_hbm.at[idx_v.at[0, pl.ds(off, R)]], sbuf, gsem)
            cp_init = pltpu.async_copy(
                feat_hbm.at[idx_v.at[1, pl.ds(off, R)]], nbuf, nsem)
            cp_init.wait()
            for j in range(2, NSLOT):
                pltpu.async_copy(
                    feat_hbm.at[idx_v.at[j, pl.ds(off, R)]], nbuf, nsem,
                    add=True).wait()
            cp_self.wait()
            pltpu.sync_copy(sbuf, self_hbm.at[pl.ds(base + off, R)])
            pltpu.sync_copy(nbuf, nsum_hbm.at[pl.ds(base + off, R)])
            return carry

        lax.fori_loop(0, nchunks, chunk, 0)

    return k(features, idxT)


def _tc_combine(self_rows, nsum, w1, w2):
    b_pad = self_rows.shape[0]

    def body(x1, x2, w1r, w2r, o):
        acc = jnp.dot(x1[...], w1r[...], preferred_element_type=jnp.float32)
        acc = acc + jnp.dot(x2[...] * jnp.float32(0.1), w2r[...],
                            preferred_element_type=jnp.float32)
        o[...] = jnp.maximum(acc, 0.0)

    return pl.pallas_call(
        body,
        grid=(b_pad // BLK,),
        in_specs=[
            pl.BlockSpec((BLK, D), lambda i: (i, 0)),
            pl.BlockSpec((BLK, D), lambda i: (i, 0)),
            pl.BlockSpec((D, D), lambda i: (0, 0)),
            pl.BlockSpec((D, D), lambda i: (0, 0)),
        ],
        out_specs=pl.BlockSpec((BLK, D), lambda i: (i, 0)),
        out_shape=jax.ShapeDtypeStruct((b_pad, D), jnp.float32),
    )(self_rows, nsum, w1, w2)


def kernel(features, weight, nodes, neigh_idx):
    b = nodes.shape[0]
    step = NW * R
    b_pad = ((b + step - 1) // step) * step

    idx_all = jnp.concatenate(
        [nodes[:, None].astype(jnp.int32), neigh_idx.astype(jnp.int32)],
        axis=1).T
    idxT = jnp.pad(idx_all, ((0, 0), (0, b_pad - b)))
    idxT = idxT.reshape(NSLOT, NW, b_pad // NW).transpose(1, 0, 2)

    self_rows, nsum = _sc_gather_sum(features, idxT, b_pad)
    out = _tc_combine(self_rows, nsum, weight[:D], weight[D:])
    return out[:b]


# speedup vs baseline: 1.3477x; 1.3477x over previous
import functools

import jax
import jax.numpy as jnp
from jax import lax
from jax.experimental import pallas as pl
from jax.experimental.pallas import tpu as pltpu
from jax.experimental.pallas import tpu_sc as plsc

D = 128
NSLOT = 11
NC, NS = 2, 16
NW = NC * NS
R = 112
BLK = 512


def _sc_gather_sum(features, idxT, b_pad):
    bpw = b_pad // NW
    nchunks = bpw // R
    mesh = plsc.VectorSubcoreMesh(core_axis_name="c", subcore_axis_name="s")

    @functools.partial(
        pl.kernel,
        out_type=(jax.ShapeDtypeStruct((b_pad, D), jnp.float32),
                  jax.ShapeDtypeStruct((b_pad, D), jnp.float32)),
        mesh=mesh,
        scratch_types=[
            pltpu.VMEM((NSLOT, bpw), jnp.int32),
            pltpu.VMEM((2, R, D), jnp.float32),
            pltpu.VMEM((2, R, D), jnp.float32),
            pltpu.SemaphoreType.DMA,
            pltpu.SemaphoreType.DMA,
            pltpu.SemaphoreType.DMA,
        ],
        compiler_params=pltpu.CompilerParams(use_tc_tiling_on_sc=False),
    )
    def k(feat_hbm, idxT_hbm, self_hbm, nsum_hbm, idx_v, sbuf, nbuf, ssem,
          nsema, nsemb):
        wid = lax.axis_index("s") * NC + lax.axis_index("c")
        base = wid * bpw
        pltpu.sync_copy(idxT_hbm.at[wid], idx_v)

        def pair(pi, carry):
            off0 = (2 * pi) * R
            off1 = off0 + R
            s0 = pltpu.async_copy(
                feat_hbm.at[idx_v.at[0, pl.ds(off0, R)]], sbuf.at[0], ssem)
            s1 = pltpu.async_copy(
                feat_hbm.at[idx_v.at[0, pl.ds(off1, R)]], sbuf.at[1], ssem)
            a = pltpu.async_copy(
                feat_hbm.at[idx_v.at[1, pl.ds(off0, R)]], nbuf.at[0], nsema)
            bcp = pltpu.async_copy(
                feat_hbm.at[idx_v.at[1, pl.ds(off1, R)]], nbuf.at[1], nsemb)
            a.wait()
            bcp.wait()
            for j in range(2, NSLOT):
                aj = pltpu.async_copy(
                    feat_hbm.at[idx_v.at[j, pl.ds(off0, R)]], nbuf.at[0],
                    nsema, add=True)
                bj = pltpu.async_copy(
                    feat_hbm.at[idx_v.at[j, pl.ds(off1, R)]], nbuf.at[1],
                    nsemb, add=True)
                aj.wait()
                bj.wait()
            s0.wait()
            s1.wait()
            pltpu.sync_copy(sbuf.at[0], self_hbm.at[pl.ds(base + off0, R)])
            pltpu.sync_copy(sbuf.at[1], self_hbm.at[pl.ds(base + off1, R)])
            pltpu.sync_copy(nbuf.at[0], nsum_hbm.at[pl.ds(base + off0, R)])
            pltpu.sync_copy(nbuf.at[1], nsum_hbm.at[pl.ds(base + off1, R)])
            return carry

        lax.fori_loop(0, nchunks // 2, pair, 0)

    return k(features, idxT)


def _tc_combine(self_rows, nsum, w1, w2):
    b_pad = self_rows.shape[0]

    def body(x1, x2, w1r, w2r, o):
        acc = jnp.dot(x1[...], w1r[...], preferred_element_type=jnp.float32)
        acc = acc + jnp.dot(x2[...] * jnp.float32(0.1), w2r[...],
                            preferred_element_type=jnp.float32)
        o[...] = jnp.maximum(acc, 0.0)

    return pl.pallas_call(
        body,
        grid=(b_pad // BLK,),
        in_specs=[
            pl.BlockSpec((BLK, D), lambda i: (i, 0)),
            pl.BlockSpec((BLK, D), lambda i: (i, 0)),
            pl.BlockSpec((D, D), lambda i: (0, 0)),
            pl.BlockSpec((D, D), lambda i: (0, 0)),
        ],
        out_specs=pl.BlockSpec((BLK, D), lambda i: (i, 0)),
        out_shape=jax.ShapeDtypeStruct((b_pad, D), jnp.float32),
    )(self_rows, nsum, w1, w2)


def kernel(features, weight, nodes, neigh_idx):
    b = nodes.shape[0]
    step = NW * R
    b_pad = ((b + step - 1) // step) * step

    idx_all = jnp.concatenate(
        [nodes[:, None].astype(jnp.int32), neigh_idx.astype(jnp.int32)],
        axis=1).T
    idxT = jnp.pad(idx_all, ((0, 0), (0, b_pad - b)))
    idxT = idxT.reshape(NSLOT, NW, b_pad // NW).transpose(1, 0, 2)

    self_rows, nsum = _sc_gather_sum(features, idxT, b_pad)
    out = _tc_combine(self_rows, nsum, weight[:D], weight[D:])
    return out[:b]
